# baseline (speedup 1.0000x reference)
import functools

import jax
import jax.numpy as jnp
from jax import lax
from jax.experimental import pallas as pl
from jax.experimental.pallas import tpu as pltpu
from jax.experimental.pallas import tpu_sc as plsc

N, E, FIN, H = 4096, 65536, 512, 512
R = 256
BE = 2048
NC, NS, L = 2, 16, 16
NW = NC * NS
B_PER_W = E // NW
C = 64
N_CHUNKS = B_PER_W // C
N_PAIRS = N_CHUNKS // 2


def _sage1_body(adj_ref, x_ref, w_ref, scale_ref, beta_ref, out_ref):
    i = pl.program_id(0)
    adj = adj_ref[...]
    agg = lax.dot_general(adj, x_ref[...], (((1,), (0,)), ((), ())),
                          preferred_element_type=jnp.float32)
    rs = jnp.sum(adj, axis=1, keepdims=True) + 1.0
    agg = agg / rs
    xblk = x_ref[pl.ds(i * R, R), :]
    h = (lax.dot_general(xblk, w_ref[0:FIN, :], (((1,), (0,)), ((), ())),
                         preferred_element_type=jnp.float32)
         + lax.dot_general(agg, w_ref[FIN:2 * FIN, :], (((1,), (0,)), ((), ())),
                           preferred_element_type=jnp.float32))
    h = jnp.maximum(h, 0.0)
    out_ref[...] = h * scale_ref[...] + beta_ref[...]


def _sage2_body(adj_ref, h1_ref, w_ref, scale_ref, beta_ref, out_ref):
    i = pl.program_id(0)
    adj = adj_ref[...]
    agg = lax.dot_general(adj, h1_ref[...], (((1,), (0,)), ((), ())),
                          preferred_element_type=jnp.float32)
    rs = jnp.sum(adj, axis=1, keepdims=True) + 1.0
    agg = agg / rs
    h1blk = h1_ref[pl.ds(i * R, R), :]
    h2 = (lax.dot_general(h1blk, w_ref[0:H, :], (((1,), (0,)), ((), ())),
                          preferred_element_type=jnp.float32)
          + lax.dot_general(agg, w_ref[H:2 * H, :], (((1,), (0,)), ((), ())),
                            preferred_element_type=jnp.float32))
    h2 = jnp.maximum(h2, 0.0)
    h2 = h2 * scale_ref[...] + beta_ref[...]
    hm = jnp.maximum(h1blk, h2)
    bl = lax.bitcast_convert_type(hm[:, 0:H2], jnp.uint32)
    bh = lax.bitcast_convert_type(hm[:, H2:H], jnp.uint32)
    rl = (bl + 0x7FFF + ((bl >> 16) & 1)) >> 16
    rh = (bh + 0x7FFF + ((bh >> 16) & 1)) & jnp.uint32(0xFFFF0000)
    out_ref[...] = rh | rl


def _mlp_body(hp_ref, p0_ref, b0_ref, p1_ref, b1_ref, p2_ref, b2_ref, out_ref):
    w = hp_ref[...]
    lo = lax.bitcast_convert_type(w << 16, jnp.float32)
    hi = lax.bitcast_convert_type(w & jnp.uint32(0xFFFF0000), jnp.float32)
    z = (lax.dot_general(lo, p0_ref[0:H2, :], (((1,), (0,)), ((), ())),
                         preferred_element_type=jnp.float32)
         + lax.dot_general(hi, p0_ref[H2:H, :], (((1,), (0,)), ((), ())),
                           preferred_element_type=jnp.float32))
    z = jnp.maximum(z + b0_ref[...], 0.0)
    z = lax.dot_general(z, p1_ref[...], (((1,), (0,)), ((), ())),
                        preferred_element_type=jnp.float32)
    z = jnp.maximum(z + b1_ref[...], 0.0)
    out_ref[...] = lax.dot_general(z, p2_ref[...], (((1,), (0,)), ((), ())),
                                   preferred_element_type=jnp.float32) + b2_ref[...]


def _full(shape):
    return pl.BlockSpec(shape, lambda i: tuple(0 for _ in shape))


def _sage_call(body, adj, hin, w, scale, beta, out_dtype=jnp.float32, out_cols=H):
    return pl.pallas_call(
        body,
        grid=(N // R,),
        in_specs=[
            pl.BlockSpec((R, N), lambda i: (i, 0)),
            _full((N, hin.shape[1])),
            _full(w.shape),
            _full((1, H)),
            _full((1, H)),
        ],
        out_specs=pl.BlockSpec((R, out_cols), lambda i: (i, 0)),
        out_shape=jax.ShapeDtypeStruct((N, out_cols), out_dtype),
        compiler_params=pltpu.CompilerParams(vmem_limit_bytes=100 * 1024 * 1024),
    )(adj, hin, w, scale, beta)


H2 = H // 2
EH = E // 8


def _make_edge_gather_mul(ne, qoff):
    b_per_w = ne // NW
    n_pairs = b_per_w // C // 2

    @functools.partial(
        pl.kernel,
        mesh=plsc.VectorSubcoreMesh(core_axis_name="c", subcore_axis_name="s"),
        out_type=jax.ShapeDtypeStruct((ne, H2), jnp.uint32),
        scratch_types=[
            pltpu.VMEM((b_per_w,), jnp.int32),
            pltpu.VMEM((b_per_w,), jnp.int32),
            pltpu.VMEM((C, H2), jnp.uint32),
            pltpu.VMEM((C, H2), jnp.uint32),
            pltpu.VMEM((C, H2), jnp.uint32),
            pltpu.VMEM((C, H2), jnp.uint32),
            pltpu.VMEM((C, H2), jnp.uint32),
            pltpu.VMEM((C, H2), jnp.uint32),
            pltpu.SemaphoreType.DMA,
            pltpu.SemaphoreType.DMA,
            pltpu.SemaphoreType.DMA,
            pltpu.SemaphoreType.DMA,
        ],
    )
    def _edge_gather_mul(edge_hbm, h_hbm, out_hbm,
                         idx_s, idx_d, sA, dA, sB, dB, prodA, prodB,
                         gA, gB, scA, scB):
        wid = lax.axis_index("s") * NC + lax.axis_index("c")
        base = qoff + wid * b_per_w

        pltpu.sync_copy(edge_hbm.at[0, pl.ds(base, b_per_w)], idx_s)
        pltpu.sync_copy(edge_hbm.at[1, pl.ds(base, b_per_w)], idx_d)
        obase = wid * b_per_w

        def gather(chunk, sbuf, dbuf, sem):
            isl = idx_s.at[pl.ds(chunk * C, C)]
            idl = idx_d.at[pl.ds(chunk * C, C)]
            pltpu.async_copy(h_hbm.at[isl], sbuf, sem)
            pltpu.async_copy(h_hbm.at[idl], dbuf, sem)

        def wait_gather(sbuf, dbuf, sem):
            pltpu.make_async_copy(h_hbm.at[idx_s.at[pl.ds(0, C)]], sbuf, sem).wait()
            pltpu.make_async_copy(h_hbm.at[idx_s.at[pl.ds(0, C)]], dbuf, sem).wait()

        def scatter(chunk, pbuf, sem):
            pltpu.async_copy(pbuf, out_hbm.at[pl.ds(obase + chunk * C, C)], sem)

        def wait_scatter(pbuf, sem):
            pltpu.make_async_copy(pbuf, out_hbm.at[pl.ds(obase, C)], sem).wait()

        def mul(sbuf, dbuf, pbuf):
            def row_body(r2, carry):
                for u in range(2):
                    r = r2 * 2 + u
                    for j in range(H2 // L):
                        sl = pl.ds(j * L, L)
                        s = sbuf[r, sl]
                        d = dbuf[r, sl]
                        pl_ = (lax.bitcast_convert_type(s << 16, jnp.float32)
                               * lax.bitcast_convert_type(d << 16, jnp.float32))
                        ph = (lax.bitcast_convert_type(s & jnp.uint32(0xFFFF0000), jnp.float32)
                              * lax.bitcast_convert_type(d & jnp.uint32(0xFFFF0000), jnp.float32))
                        bl = lax.bitcast_convert_type(pl_, jnp.uint32)
                        bh = lax.bitcast_convert_type(ph, jnp.uint32)
                        rl = (bl + 0x8000) >> 16
                        rh = (bh + 0x8000) & jnp.uint32(0xFFFF0000)
                        pbuf[r, sl] = rh | rl
                return carry
            lax.fori_loop(0, C // 2, row_body, 0)

        gather(0, sA, dA, gA)

        def pair_body(p, carry):
            a = 2 * p
            b = a + 1

            @pl.when(p > 0)
            def _():
                wait_scatter(prodB, scB)
            gather(b, sB, dB, gB)
            wait_gather(sA, dA, gA)

            @pl.when(p > 0)
            def _():
                wait_scatter(prodA, scA)
            mul(sA, dA, prodA)
            scatter(a, prodA, scA)

            @pl.when(p < n_pairs - 1)
            def _():
                gather(a + 2, sA, dA, gA)
            wait_gather(sB, dB, gB)
            mul(sB, dB, prodB)
            scatter(b, prodB, scB)
            return carry

        lax.fori_loop(0, n_pairs, pair_body, 0)
        wait_scatter(prodA, scA)
        wait_scatter(prodB, scB)

    return _edge_gather_mul


def _mlp_call(hp, P0w, P0b, P1w, P1b, P2w, P2b):
    ne = hp.shape[0]
    return pl.pallas_call(
        _mlp_body,
        grid=(ne // BE,),
        in_specs=[
            pl.BlockSpec((BE, H2), lambda i: (i, 0)),
            _full((H, H)),
            _full((1, H)),
            _full((H, H)),
            _full((1, H)),
            _full((H, 2)),
            _full((1, 2)),
        ],
        out_specs=pl.BlockSpec((BE, 2), lambda i: (i, 0)),
        out_shape=jax.ShapeDtypeStruct((ne, 2), jnp.float32),
        compiler_params=pltpu.CompilerParams(vmem_limit_bytes=100 * 1024 * 1024),
    )(hp, P0w, P0b.reshape(1, H), P1w, P1b.reshape(1, H), P2w, P2b.reshape(1, 2))


def kernel(edge_index, adj, x, y, W0, W1, W2, gamma, beta,
           P0w, P0b, P1w, P1b, P2w, P2b):
    scale = (gamma / jnp.sqrt(1.0 + 1e-5)).reshape(1, H)
    beta2 = beta.reshape(1, H)

    h1 = _sage_call(_sage1_body, adj, x, W0, scale, beta2)
    h = _sage_call(_sage2_body, adj, h1, W1, scale, beta2,
                   out_dtype=jnp.uint32, out_cols=H2)

    hps = [_make_edge_gather_mul(EH, i * EH)(edge_index, h)
           for i in range(E // EH)]
    outs = [_mlp_call(hp, P0w, P0b, P1w, P1b, P2w, P2b) for hp in hps]
    out = jnp.concatenate(outs, axis=0)

    return (out, y)


# speedup vs baseline: 1.1513x; 1.1513x over previous
import functools

import jax
import jax.numpy as jnp
from jax import lax
from jax.experimental import pallas as pl
from jax.experimental.pallas import tpu as pltpu
from jax.experimental.pallas import tpu_sc as plsc

N, E, FIN, H = 4096, 65536, 512, 512
R = 256
BE = 2048
NC, NS, L = 2, 16, 16
NW = NC * NS
B_PER_W = E // NW
C = 64
N_CHUNKS = B_PER_W // C
N_PAIRS = N_CHUNKS // 2


def _sage1_body(adj_ref, x_ref, w_ref, scale_ref, beta_ref, out_ref):
    i = pl.program_id(0)
    adj = adj_ref[...]
    agg = lax.dot_general(adj, x_ref[...], (((1,), (0,)), ((), ())),
                          preferred_element_type=jnp.float32)
    rs = jnp.sum(adj, axis=1, keepdims=True) + 1.0
    agg = agg / rs
    xblk = x_ref[pl.ds(i * R, R), :]
    h = (lax.dot_general(xblk, w_ref[0:FIN, :], (((1,), (0,)), ((), ())),
                         preferred_element_type=jnp.float32)
         + lax.dot_general(agg, w_ref[FIN:2 * FIN, :], (((1,), (0,)), ((), ())),
                           preferred_element_type=jnp.float32))
    h = jnp.maximum(h, 0.0)
    out_ref[...] = h * scale_ref[...] + beta_ref[...]


def _sage2_body(adj_ref, h1_ref, w_ref, scale_ref, beta_ref, out_ref):
    i = pl.program_id(0)
    adj = adj_ref[...]
    agg = lax.dot_general(adj, h1_ref[...], (((1,), (0,)), ((), ())),
                          preferred_element_type=jnp.float32)
    rs = jnp.sum(adj, axis=1, keepdims=True) + 1.0
    agg = agg / rs
    h1blk = h1_ref[pl.ds(i * R, R), :]
    h2 = (lax.dot_general(h1blk, w_ref[0:H, :], (((1,), (0,)), ((), ())),
                          preferred_element_type=jnp.float32)
          + lax.dot_general(agg, w_ref[H:2 * H, :], (((1,), (0,)), ((), ())),
                            preferred_element_type=jnp.float32))
    h2 = jnp.maximum(h2, 0.0)
    h2 = h2 * scale_ref[...] + beta_ref[...]
    hm = jnp.maximum(h1blk, h2)
    bl = lax.bitcast_convert_type(hm[:, 0:H2], jnp.uint32)
    bh = lax.bitcast_convert_type(hm[:, H2:H], jnp.uint32)
    rl = (bl + 0x7FFF + ((bl >> 16) & 1)) >> 16
    rh = (bh + 0x7FFF + ((bh >> 16) & 1)) & jnp.uint32(0xFFFF0000)
    out_ref[...] = rh | rl


def _mlp_body(hp_ref, p0_ref, b0_ref, p1_ref, b1_ref, p2_ref, b2_ref, out_ref):
    w = hp_ref[...]
    lo = lax.bitcast_convert_type(w << 16, jnp.float32)
    hi = lax.bitcast_convert_type(w & jnp.uint32(0xFFFF0000), jnp.float32)
    z = (lax.dot_general(lo, p0_ref[0:H2, :], (((1,), (0,)), ((), ())),
                         preferred_element_type=jnp.float32)
         + lax.dot_general(hi, p0_ref[H2:H, :], (((1,), (0,)), ((), ())),
                           preferred_element_type=jnp.float32))
    z = jnp.maximum(z + b0_ref[...], 0.0)
    z = lax.dot_general(z, p1_ref[...], (((1,), (0,)), ((), ())),
                        preferred_element_type=jnp.float32)
    z = jnp.maximum(z + b1_ref[...], 0.0)
    out_ref[...] = lax.dot_general(z, p2_ref[...], (((1,), (0,)), ((), ())),
                                   preferred_element_type=jnp.float32) + b2_ref[...]


def _full(shape):
    return pl.BlockSpec(shape, lambda i: tuple(0 for _ in shape))


def _sage_call(body, adj, hin, w, scale, beta, out_dtype=jnp.float32, out_cols=H):
    return pl.pallas_call(
        body,
        grid=(N // R,),
        in_specs=[
            pl.BlockSpec((R, N), lambda i: (i, 0)),
            _full((N, hin.shape[1])),
            _full(w.shape),
            _full((1, H)),
            _full((1, H)),
        ],
        out_specs=pl.BlockSpec((R, out_cols), lambda i: (i, 0)),
        out_shape=jax.ShapeDtypeStruct((N, out_cols), out_dtype),
        compiler_params=pltpu.CompilerParams(vmem_limit_bytes=100 * 1024 * 1024),
    )(adj, hin, w, scale, beta)


H2 = H // 2
EH = E // 4


def _make_edge_gather_mul(ne, qoff):
    b_per_w = ne // NW
    n_pairs = b_per_w // C // 2

    @functools.partial(
        pl.kernel,
        mesh=plsc.VectorSubcoreMesh(core_axis_name="c", subcore_axis_name="s"),
        out_type=jax.ShapeDtypeStruct((ne, H2), jnp.uint32),
        scratch_types=[
            pltpu.VMEM((b_per_w,), jnp.int32),
            pltpu.VMEM((b_per_w,), jnp.int32),
            pltpu.VMEM((C, H2), jnp.uint32),
            pltpu.VMEM((C, H2), jnp.uint32),
            pltpu.VMEM((C, H2), jnp.uint32),
            pltpu.VMEM((C, H2), jnp.uint32),
            pltpu.VMEM((C, H2), jnp.uint32),
            pltpu.VMEM((C, H2), jnp.uint32),
            pltpu.SemaphoreType.DMA,
            pltpu.SemaphoreType.DMA,
            pltpu.SemaphoreType.DMA,
            pltpu.SemaphoreType.DMA,
        ],
    )
    def _edge_gather_mul(edge_hbm, h_hbm, out_hbm,
                         idx_s, idx_d, sA, dA, sB, dB, prodA, prodB,
                         gA, gB, scA, scB):
        wid = lax.axis_index("s") * NC + lax.axis_index("c")
        base = qoff + wid * b_per_w

        pltpu.sync_copy(edge_hbm.at[0, pl.ds(base, b_per_w)], idx_s)
        pltpu.sync_copy(edge_hbm.at[1, pl.ds(base, b_per_w)], idx_d)
        obase = wid * b_per_w

        def gather(chunk, sbuf, dbuf, sem):
            isl = idx_s.at[pl.ds(chunk * C, C)]
            idl = idx_d.at[pl.ds(chunk * C, C)]
            pltpu.async_copy(h_hbm.at[isl], sbuf, sem)
            pltpu.async_copy(h_hbm.at[idl], dbuf, sem)

        def wait_gather(sbuf, dbuf, sem):
            pltpu.make_async_copy(h_hbm.at[idx_s.at[pl.ds(0, C)]], sbuf, sem).wait()
            pltpu.make_async_copy(h_hbm.at[idx_s.at[pl.ds(0, C)]], dbuf, sem).wait()

        def scatter(chunk, pbuf, sem):
            pltpu.async_copy(pbuf, out_hbm.at[pl.ds(obase + chunk * C, C)], sem)

        def wait_scatter(pbuf, sem):
            pltpu.make_async_copy(pbuf, out_hbm.at[pl.ds(obase, C)], sem).wait()

        def mul(sbuf, dbuf, pbuf):
            def row_body(r2, carry):
                for u in range(2):
                    r = r2 * 2 + u
                    for j in range(H2 // L):
                        sl = pl.ds(j * L, L)
                        s = sbuf[r, sl]
                        d = dbuf[r, sl]
                        pl_ = (lax.bitcast_convert_type(s << 16, jnp.float32)
                               * lax.bitcast_convert_type(d << 16, jnp.float32))
                        ph = (lax.bitcast_convert_type(s & jnp.uint32(0xFFFF0000), jnp.float32)
                              * lax.bitcast_convert_type(d & jnp.uint32(0xFFFF0000), jnp.float32))
                        bl = lax.bitcast_convert_type(pl_, jnp.uint32)
                        bh = lax.bitcast_convert_type(ph, jnp.uint32)
                        rl = (bl + 0x8000) >> 16
                        rh = (bh + 0x8000) & jnp.uint32(0xFFFF0000)
                        pbuf[r, sl] = rh | rl
                return carry
            lax.fori_loop(0, C // 2, row_body, 0)

        gather(0, sA, dA, gA)

        def pair_body(p, carry):
            a = 2 * p
            b = a + 1

            @pl.when(p > 0)
            def _():
                wait_scatter(prodB, scB)
            gather(b, sB, dB, gB)
            wait_gather(sA, dA, gA)

            @pl.when(p > 0)
            def _():
                wait_scatter(prodA, scA)
            mul(sA, dA, prodA)
            scatter(a, prodA, scA)

            @pl.when(p < n_pairs - 1)
            def _():
                gather(a + 2, sA, dA, gA)
            wait_gather(sB, dB, gB)
            mul(sB, dB, prodB)
            scatter(b, prodB, scB)
            return carry

        lax.fori_loop(0, n_pairs, pair_body, 0)
        wait_scatter(prodA, scA)
        wait_scatter(prodB, scB)

    return _edge_gather_mul


def _mlp_call(hp, P0w, P0b, P1w, P1b, P2w, P2b):
    ne = hp.shape[0]
    return pl.pallas_call(
        _mlp_body,
        grid=(ne // BE,),
        in_specs=[
            pl.BlockSpec((BE, H2), lambda i: (i, 0)),
            _full((H, H)),
            _full((1, H)),
            _full((H, H)),
            _full((1, H)),
            _full((H, 2)),
            _full((1, 2)),
        ],
        out_specs=pl.BlockSpec((BE, 2), lambda i: (i, 0)),
        out_shape=jax.ShapeDtypeStruct((ne, 2), jnp.float32),
        compiler_params=pltpu.CompilerParams(vmem_limit_bytes=100 * 1024 * 1024),
    )(hp, P0w, P0b.reshape(1, H), P1w, P1b.reshape(1, H), P2w, P2b.reshape(1, 2))


def kernel(edge_index, adj, x, y, W0, W1, W2, gamma, beta,
           P0w, P0b, P1w, P1b, P2w, P2b):
    scale = (gamma / jnp.sqrt(1.0 + 1e-5)).reshape(1, H)
    beta2 = beta.reshape(1, H)

    h1 = _sage_call(_sage1_body, adj, x, W0, scale, beta2)
    h = _sage_call(_sage2_body, adj, h1, W1, scale, beta2,
                   out_dtype=jnp.uint32, out_cols=H2)

    hps = [_make_edge_gather_mul(EH, i * EH)(edge_index, h)
           for i in range(E // EH)]
    outs = [_mlp_call(hp, P0w, P0b, P1w, P1b, P2w, P2b) for hp in hps]
    out = jnp.concatenate(outs, axis=0)

    return (out, y)
